# Initial kernel scaffold; baseline (speedup 1.0000x reference)
import jax
import jax.numpy as jnp
from jax.experimental import pallas as pl


def kernel(top_indices, distance_emb):
    raise NotImplementedError("write your pallas kernel here")


# speedup vs baseline: 1.2166x; 1.2166x over previous
import functools

import jax
import jax.numpy as jnp
from jax import lax
from jax.experimental import pallas as pl
from jax.experimental.pallas import tpu as pltpu
from jax.experimental.pallas import tpu_sc as plsc

N_WORDS = 8192
TOP_K = 50
EMB = 64

NC = 2
NS = 16
L = 16
NW = NC * NS
TOTAL = N_WORDS * TOP_K
PER_TILE = TOTAL // NW
GROUPS = PER_TILE // L

CHUNK = 128
NCHUNK = PER_TILE // CHUNK
NBUF = 6
LOOKAHEAD = NBUF - 2


def _body(ti_hbm, emb_hbm, out_hbm, idx_v, bufs, gsems, wsems):
    wid = lax.axis_index("s") * NC + lax.axis_index("c")
    base = wid * PER_TILE

    pltpu.sync_copy(ti_hbm.at[pl.ds(base, PER_TILE)], idx_v)

    lanes = lax.iota(jnp.int32, L)

    def g_body(g, carry):
        t = idx_v[pl.ds(g * L, L)]
        p = base + g * L + lanes
        word = lax.div(p, jnp.int32(TOP_K))
        d = jnp.maximum(word - t, 1)
        b = jnp.minimum(jnp.maximum(d - 1, 0), 1)
        for thr in (3, 4, 5, 8, 16, 32, 64):
            b = b + jnp.minimum(jnp.maximum(d - (thr - 1), 0), 1)
        idx_v[pl.ds(g * L, L)] = b
        return carry

    lax.fori_loop(0, GROUPS, g_body, 0)

    gd = [None] * NCHUNK
    wd = [None] * NCHUNK

    def start_gather(c):
        gd[c] = pltpu.async_copy(
            emb_hbm.at[idx_v.at[pl.ds(c * CHUNK, CHUNK)]],
            bufs[c % NBUF], gsems[c % NBUF])

    for c in range(min(LOOKAHEAD, NCHUNK)):
        start_gather(c)
    for c in range(NCHUNK):
        nxt = c + LOOKAHEAD
        if nxt < NCHUNK:
            prev = nxt - NBUF
            if prev >= 0:
                wd[prev].wait()
            start_gather(nxt)
        gd[c].wait()
        wd[c] = pltpu.async_copy(
            bufs[c % NBUF],
            out_hbm.at[pl.ds(base + c * CHUNK, CHUNK)],
            wsems[c % NBUF])
    for c in range(max(0, NCHUNK - NBUF), NCHUNK):
        wd[c].wait()


def _sc_call(ti_flat, emb):
    mesh = plsc.VectorSubcoreMesh(
        core_axis_name="c", subcore_axis_name="s",
        num_cores=NC, num_subcores=NS)

    def body(ti_hbm, emb_hbm, out_hbm, idx_v, *rest):
        bufs = rest[:NBUF]
        gsems = rest[NBUF:2 * NBUF]
        wsems = rest[2 * NBUF:]
        _body(ti_hbm, emb_hbm, out_hbm, idx_v, bufs, gsems, wsems)

    scratch = (
        [pltpu.VMEM((PER_TILE,), jnp.int32)]
        + [pltpu.VMEM((CHUNK, EMB), jnp.float32) for _ in range(NBUF)]
        + [pltpu.SemaphoreType.DMA for _ in range(2 * NBUF)]
    )
    k = pl.kernel(
        body,
        out_type=jax.ShapeDtypeStruct((TOTAL, EMB), jnp.float32),
        mesh=mesh,
        scratch_types=scratch,
        compiler_params=pltpu.CompilerParams(use_tc_tiling_on_sc=False),
    )
    return k(ti_flat, emb)


def kernel(top_indices, distance_emb):
    ti_flat = top_indices.reshape(-1).astype(jnp.int32)
    out = _sc_call(ti_flat, distance_emb)
    return out.reshape(N_WORDS, TOP_K, EMB)


# speedup vs baseline: 12.7197x; 10.4550x over previous
import functools

import jax
import jax.numpy as jnp
from jax import lax
from jax.experimental import pallas as pl
from jax.experimental.pallas import tpu as pltpu
from jax.experimental.pallas import tpu_sc as plsc

N_WORDS = 8192
TOP_K = 50
EMB = 64

NC = 2
NS = 16
L = 16
NW = NC * NS
TOTAL = N_WORDS * TOP_K
PER_TILE = TOTAL // NW
GROUPS = PER_TILE // L

CHUNK = 128
NCHUNK = PER_TILE // CHUNK
NBUF = 6
LOOKAHEAD = NBUF - 2


def _body(ti_hbm, emb_hbm, out_hbm, emb_sh, idx_v, bufs, gsems, wsems):
    cid = lax.axis_index("c")
    sid = lax.axis_index("s")
    wid = sid * NC + cid
    base = wid * PER_TILE

    @pl.when(sid == 0)
    def _():
        pltpu.sync_copy(emb_hbm, emb_sh)

    pltpu.sync_copy(ti_hbm.at[pl.ds(base, PER_TILE)], idx_v)
    plsc.subcore_barrier()

    lanes = lax.iota(jnp.int32, L)

    def g_body(g, carry):
        t = idx_v[pl.ds(g * L, L)]
        p = base + g * L + lanes
        word = lax.div(p, jnp.int32(TOP_K))
        d = jnp.maximum(word - t, 1)
        b = jnp.minimum(jnp.maximum(d - 1, 0), 1)
        for thr in (3, 4, 5, 8, 16, 32, 64):
            b = b + jnp.minimum(jnp.maximum(d - (thr - 1), 0), 1)
        idx_v[pl.ds(g * L, L)] = b
        return carry

    lax.fori_loop(0, GROUPS, g_body, 0)

    gd = [None] * NCHUNK
    wd = [None] * NCHUNK

    def start_gather(c):
        gd[c] = pltpu.async_copy(
            emb_sh.at[idx_v.at[pl.ds(c * CHUNK, CHUNK)]],
            bufs[c % NBUF], gsems[c % NBUF])

    for c in range(min(LOOKAHEAD, NCHUNK)):
        start_gather(c)
    for c in range(NCHUNK):
        nxt = c + LOOKAHEAD
        if nxt < NCHUNK:
            prev = nxt - NBUF
            if prev >= 0:
                wd[prev].wait()
            start_gather(nxt)
        gd[c].wait()
        wd[c] = pltpu.async_copy(
            bufs[c % NBUF],
            out_hbm.at[pl.ds(base + c * CHUNK, CHUNK)],
            wsems[c % NBUF])
    for c in range(max(0, NCHUNK - NBUF), NCHUNK):
        wd[c].wait()


def _sc_call(ti_flat, emb):
    mesh = plsc.VectorSubcoreMesh(
        core_axis_name="c", subcore_axis_name="s",
        num_cores=NC, num_subcores=NS)

    def body(ti_hbm, emb_hbm, out_hbm, emb_sh, idx_v, *rest):
        bufs = rest[:NBUF]
        gsems = rest[NBUF:2 * NBUF]
        wsems = rest[2 * NBUF:]
        _body(ti_hbm, emb_hbm, out_hbm, emb_sh, idx_v, bufs, gsems, wsems)

    scratch = (
        [pltpu.VMEM_SHARED((9, EMB), jnp.float32)]
        + [pltpu.VMEM((PER_TILE,), jnp.int32)]
        + [pltpu.VMEM((CHUNK, EMB), jnp.float32) for _ in range(NBUF)]
        + [pltpu.SemaphoreType.DMA for _ in range(2 * NBUF)]
    )
    k = pl.kernel(
        body,
        out_type=jax.ShapeDtypeStruct((TOTAL, EMB), jnp.float32),
        mesh=mesh,
        scratch_types=scratch,
        compiler_params=pltpu.CompilerParams(use_tc_tiling_on_sc=False),
    )
    return k(ti_flat, emb)


def kernel(top_indices, distance_emb):
    ti_flat = top_indices.reshape(-1).astype(jnp.int32)
    out = _sc_call(ti_flat, distance_emb)
    return out.reshape(N_WORDS, TOP_K, EMB)
